# Initial kernel scaffold; baseline (speedup 1.0000x reference)
import jax
import jax.numpy as jnp
from jax.experimental import pallas as pl


def kernel(indices, table):
    raise NotImplementedError("write your pallas kernel here")


# speedup vs baseline: 1.5614x; 1.5614x over previous
import functools

import jax
import jax.numpy as jnp
from jax import lax
from jax.experimental import pallas as pl
from jax.experimental.pallas import tpu as pltpu
from jax.experimental.pallas import tpu_sc as plsc

D = 64
BATCH = 4096
SEQ = 200
SUB = 3
T = BATCH * SEQ
NC = 2
NS = 16
NW = NC * NS
TPW = T // NW
C = 256
IDX_PER_CHUNK = SUB * C
G = 128
NG = IDX_PER_CHUNK // G
NCHUNK = TPW // C


def _make_kernel():
    mesh = plsc.VectorSubcoreMesh(core_axis_name="c", subcore_axis_name="s")

    @functools.partial(
        pl.kernel,
        mesh=mesh,
        out_type=jax.ShapeDtypeStruct((T, D), jnp.float32),
        compiler_params=pltpu.CompilerParams(use_tc_tiling_on_sc=False),
        scratch_types=[
            pltpu.VMEM((IDX_PER_CHUNK,), jnp.int32),
            pltpu.VMEM((IDX_PER_CHUNK, D), jnp.float32),
            pltpu.VMEM((C, D), jnp.float32),
            pltpu.SemaphoreType.DMA,
        ],
    )
    def emb_kernel(table_hbm, idx_hbm, out_hbm, idx_v, rows_v, out_v, sem):
        wid = lax.axis_index("s") * NC + lax.axis_index("c")
        tok0 = wid * TPW

        def chunk_body(c, carry):
            base_tok = tok0 + c * C
            pltpu.sync_copy(
                idx_hbm.at[pl.ds(base_tok * SUB, IDX_PER_CHUNK)], idx_v
            )
            copies = [
                pltpu.async_copy(
                    table_hbm.at[idx_v.at[pl.ds(j * G, G)]],
                    rows_v.at[pl.ds(j * G, G)],
                    sem,
                )
                for j in range(NG)
            ]
            for cp in copies:
                cp.wait()

            def token_body(t, _):
                r = SUB * t
                for d in range(D // 16):
                    sl = pl.ds(d * 16, 16)
                    out_v[t, sl] = (
                        rows_v[r, sl] + rows_v[r + 1, sl] + rows_v[r + 2, sl]
                    )
                return _

            lax.fori_loop(0, C, token_body, 0)
            pltpu.sync_copy(out_v, out_hbm.at[pl.ds(base_tok, C)])
            return carry

        lax.fori_loop(0, NCHUNK, chunk_body, 0)

    return emb_kernel


_emb = _make_kernel()


def kernel(indices, table):
    idx_flat = indices.reshape(-1).astype(jnp.int32)
    out = _emb(table, idx_flat)
    return out.reshape(BATCH, SEQ, D)


# speedup vs baseline: 1.6606x; 1.0635x over previous
import functools

import jax
import jax.numpy as jnp
from jax import lax
from jax.experimental import pallas as pl
from jax.experimental.pallas import tpu as pltpu
from jax.experimental.pallas import tpu_sc as plsc

D = 64
BATCH = 4096
SEQ = 200
SUB = 3
T = BATCH * SEQ
NC = 2
NS = 16
NW = NC * NS
TPW = T // NW
C = 128
IDX_PER_CHUNK = SUB * C
G = 128
NG = IDX_PER_CHUNK // G
NCHUNK = TPW // C


def _make_kernel():
    mesh = plsc.VectorSubcoreMesh(core_axis_name="c", subcore_axis_name="s")

    @functools.partial(
        pl.kernel,
        mesh=mesh,
        out_type=jax.ShapeDtypeStruct((T, D), jnp.float32),
        compiler_params=pltpu.CompilerParams(use_tc_tiling_on_sc=False),
        scratch_types=[
            pltpu.VMEM((IDX_PER_CHUNK,), jnp.int32),
            pltpu.VMEM((IDX_PER_CHUNK,), jnp.int32),
            pltpu.VMEM((IDX_PER_CHUNK, D), jnp.float32),
            pltpu.VMEM((IDX_PER_CHUNK, D), jnp.float32),
            pltpu.VMEM((C, D), jnp.float32),
            pltpu.VMEM((C, D), jnp.float32),
            pltpu.SemaphoreType.DMA,
            pltpu.SemaphoreType.DMA,
            pltpu.SemaphoreType.DMA,
            pltpu.SemaphoreType.DMA,
        ],
    )
    def emb_kernel(table_hbm, idx_hbm, out_hbm,
                   idx0, idx1, rows0, rows1, acc0, acc1,
                   sg0, sg1, so0, so1):
        idx_v = (idx0, idx1)
        rows_v = (rows0, rows1)
        out_v = (acc0, acc1)
        sg = (sg0, sg1)
        so = (so0, so1)
        wid = lax.axis_index("s") * NC + lax.axis_index("c")
        tok0 = wid * TPW

        def stage(c, b):
            pltpu.sync_copy(
                idx_hbm.at[pl.ds((tok0 + c * C) * SUB, IDX_PER_CHUNK)],
                idx_v[b],
            )
            for j in range(NG):
                pltpu.async_copy(
                    table_hbm.at[idx_v[b].at[pl.ds(j * G, G)]],
                    rows_v[b].at[pl.ds(j * G, G)],
                    sg[b],
                )

        stage(0, 0)
        stage(1, 1)

        def outer(g, carry):
            for b in range(2):
                c = 2 * g + b
                base_tok = tok0 + c * C
                pltpu.make_async_copy(
                    table_hbm.at[pl.ds(0, IDX_PER_CHUNK)], rows_v[b], sg[b]
                ).wait()

                @pl.when(c >= 2)
                def _():
                    pltpu.make_async_copy(
                        out_v[b], out_hbm.at[pl.ds(tok0, C)], so[b]
                    ).wait()

                def token_body(t, _):
                    r = SUB * t
                    for d in range(D // 16):
                        sl = pl.ds(d * 16, 16)
                        out_v[b][t, sl] = (
                            rows_v[b][r, sl]
                            + rows_v[b][r + 1, sl]
                            + rows_v[b][r + 2, sl]
                        )
                    return _

                lax.fori_loop(0, C, token_body, 0)
                pltpu.async_copy(
                    out_v[b], out_hbm.at[pl.ds(base_tok, C)], so[b]
                )

                @pl.when(c + 2 < NCHUNK)
                def _():
                    stage(c + 2, b)

            return carry

        lax.fori_loop(0, NCHUNK // 2, outer, 0)
        for b in range(2):
            pltpu.make_async_copy(
                out_v[b], out_hbm.at[pl.ds(tok0, C)], so[b]
            ).wait()

    return emb_kernel


_emb = _make_kernel()


def kernel(indices, table):
    idx_flat = indices.reshape(-1).astype(jnp.int32)
    out = _emb(table, idx_flat)
    return out.reshape(BATCH, SEQ, D)
